# baseline (speedup 1.0000x reference)
import jax
import jax.numpy as jnp
from jax.experimental import pallas as pl
from jax.experimental.pallas import tpu as pltpu


def _dot_t(a, b):
    return jax.lax.dot_general(
        a, b, (((1,), (1,)), ((), ())), preferred_element_type=jnp.float32
    )


def kernel(white_features, black_features, turn, score, result, W0, b0, W1, b1, W2, b2):
    B, F = white_features.shape
    M = W0.shape[0]
    N = W1.shape[0]
    BB = 16
    NB = B // BB

    w2r = W2.reshape(1, N)
    b0b0r = jnp.concatenate([b0, b0]).reshape(1, 2 * M)
    b1r_ = b1.reshape(1, N)
    b2r_ = b2.reshape(1, 1)

    def outer(white_hbm, black_hbm, turn_hbm, score_hbm, w0_ref, w1_ref,
              w2_ref, b0_ref, b1_ref, b2_ref, out_hbm):

        def inner(white_ref, black_ref, turn_ref, score_ref, out_ref):
            wpT = _dot_t(w0_ref[...], white_ref[...])
            bpT = _dot_t(w0_ref[...], black_ref[...])
            a = jnp.concatenate([wpT, bpT], axis=0).T + b0_ref[...]
            swapped = jnp.concatenate([a[:, M:], a[:, :M]], axis=1)
            t = turn_ref[...]
            accum = t * a + (1.0 - t) * swapped
            l1 = jnp.clip(accum, 0.0, 1.0)
            l2 = jnp.clip(_dot_t(l1, w1_ref[...]) + b1_ref[...], 0.0, 1.0)
            model_result = jnp.sum(l2 * w2_ref[...], axis=1, keepdims=True) + b2_ref[...]
            wdl_model = jax.nn.sigmoid(model_result / 400.0)
            wdl_target = jax.nn.sigmoid(score_ref[...] / 400.0)
            out_ref[...] = (wdl_model - wdl_target) ** 2

        pipe = pltpu.emit_pipeline(
            inner,
            grid=(NB,),
            in_specs=[
                pl.BlockSpec((BB, F), lambda j: (j, 0),
                             pipeline_mode=pl.Buffered(buffer_count=3)),
                pl.BlockSpec((BB, F), lambda j: (j, 0),
                             pipeline_mode=pl.Buffered(buffer_count=3)),
                pl.BlockSpec((BB, 1), lambda j: (j, 0)),
                pl.BlockSpec((BB, 1), lambda j: (j, 0)),
            ],
            out_specs=[pl.BlockSpec((BB, 1), lambda j: (j, 0))],
        )
        pipe(white_hbm, black_hbm, turn_hbm, score_hbm, out_hbm)

    loss = pl.pallas_call(
        outer,
        in_specs=[
            pl.BlockSpec(memory_space=pl.ANY),
            pl.BlockSpec(memory_space=pl.ANY),
            pl.BlockSpec(memory_space=pl.ANY),
            pl.BlockSpec(memory_space=pl.ANY),
            pl.BlockSpec((M, F), lambda: (0, 0)),
            pl.BlockSpec(W1.shape, lambda: (0, 0)),
            pl.BlockSpec((1, N), lambda: (0, 0)),
            pl.BlockSpec((1, 2 * M), lambda: (0, 0)),
            pl.BlockSpec((1, N), lambda: (0, 0)),
            pl.BlockSpec((1, 1), lambda: (0, 0)),
        ],
        out_specs=pl.BlockSpec(memory_space=pl.ANY),
        out_shape=jax.ShapeDtypeStruct((B, 1), jnp.float32),
    )(white_features, black_features, turn, score, W0, W1, w2r, b0b0r,
      b1r_, b2r_)
    return loss


# speedup vs baseline: 1.0247x; 1.0247x over previous
import jax
import jax.numpy as jnp
from jax.experimental import pallas as pl
from jax.experimental.pallas import tpu as pltpu


def _dot_t(a, b):
    return jax.lax.dot_general(
        a, b, (((1,), (1,)), ((), ())), preferred_element_type=jnp.float32
    )


def kernel(white_features, black_features, turn, score, result, W0, b0, W1, b1, W2, b2):
    B, F = white_features.shape
    M = W0.shape[0]
    N = W1.shape[0]
    BB = 32
    NB = B // BB

    w2r = W2.reshape(1, N)
    b0b0r = jnp.concatenate([b0, b0]).reshape(1, 2 * M)
    b1r_ = b1.reshape(1, N)
    b2r_ = b2.reshape(1, 1)

    def body(white_ref, black_ref, w0_ref, w1_ref, w2_ref, b0_ref, b1_ref, b2_ref, turn_ref, score_ref, out_ref):
        j = pl.program_id(0)
        rows = pl.ds(j * BB, BB)
        wpT = _dot_t(w0_ref[...], white_ref[...])
        bpT = _dot_t(w0_ref[...], black_ref[...])
        w1 = w1_ref[...]
        w2 = w2_ref[...]
        b0b0 = b0_ref[...]
        b1r = b1_ref[...]
        b2s = b2_ref[...]
        a = jnp.concatenate([wpT, bpT], axis=0).T + b0b0
        swapped = jnp.concatenate([a[:, M:], a[:, :M]], axis=1)
        t = turn_ref[rows, :]
        accum = t * a + (1.0 - t) * swapped
        l1 = jnp.clip(accum, 0.0, 1.0)
        l2 = jnp.clip(_dot_t(l1, w1) + b1r, 0.0, 1.0)
        model_result = jnp.sum(l2 * w2, axis=1, keepdims=True) + b2s
        wdl_model = jax.nn.sigmoid(model_result / 400.0)
        wdl_target = jax.nn.sigmoid(score_ref[rows, :] / 400.0)
        out_ref[rows, :] = (wdl_model - wdl_target) ** 2

    loss = pl.pallas_call(
        body,
        grid=(NB,),
        in_specs=[
            pl.BlockSpec((BB, F), lambda j: (j, 0)),
            pl.BlockSpec((BB, F), lambda j: (j, 0)),
            pl.BlockSpec((M, F), lambda j: (0, 0)),
            pl.BlockSpec(W1.shape, lambda j: (0, 0)),
            pl.BlockSpec((1, N), lambda j: (0, 0)),
            pl.BlockSpec((1, 2 * M), lambda j: (0, 0)),
            pl.BlockSpec((1, N), lambda j: (0, 0)),
            pl.BlockSpec((1, 1), lambda j: (0, 0)),
            pl.BlockSpec((B, 1), lambda j: (0, 0)),
            pl.BlockSpec((B, 1), lambda j: (0, 0)),
        ],
        out_specs=pl.BlockSpec((B, 1), lambda j: (0, 0)),
        out_shape=jax.ShapeDtypeStruct((B, 1), jnp.float32),
        compiler_params=pltpu.CompilerParams(
            dimension_semantics=("arbitrary",),
        ),
    )(white_features, black_features, W0, W1, w2r, b0b0r, b1r_, b2r_, turn, score)
    return loss


# speedup vs baseline: 1.0369x; 1.0119x over previous
import jax
import jax.numpy as jnp
from jax.experimental import pallas as pl
from jax.experimental.pallas import tpu as pltpu


def _dot_t(a, b):
    return jax.lax.dot_general(
        a, b, (((1,), (1,)), ((), ())), preferred_element_type=jnp.float32
    )


def kernel(white_features, black_features, turn, score, result, W0, b0, W1, b1, W2, b2):
    B, F = white_features.shape
    M = W0.shape[0]
    N = W1.shape[0]
    BB = 32
    NB = B // BB

    w2r = W2.reshape(1, N)
    b0b0r = jnp.concatenate([b0, b0]).reshape(1, 2 * M)
    b1r_ = b1.reshape(1, N)
    b2r_ = b2.reshape(1, 1)

    def outer(white_hbm, black_hbm, turn_hbm, score_hbm, w0_ref, w1_ref,
              w2_ref, b0_ref, b1_ref, b2_ref, out_hbm):

        def inner(white_ref, black_ref, turn_ref, score_ref, out_ref):
            wpT = _dot_t(w0_ref[...], white_ref[...])
            bpT = _dot_t(w0_ref[...], black_ref[...])
            a = jnp.concatenate([wpT, bpT], axis=0).T + b0_ref[...]
            swapped = jnp.concatenate([a[:, M:], a[:, :M]], axis=1)
            t = turn_ref[...]
            accum = t * a + (1.0 - t) * swapped
            l1 = jnp.clip(accum, 0.0, 1.0)
            l2 = jnp.clip(_dot_t(l1, w1_ref[...]) + b1_ref[...], 0.0, 1.0)
            model_result = jnp.sum(l2 * w2_ref[...], axis=1, keepdims=True) + b2_ref[...]
            wdl_model = jax.nn.sigmoid(model_result / 400.0)
            wdl_target = jax.nn.sigmoid(score_ref[...] / 400.0)
            out_ref[...] = (wdl_model - wdl_target) ** 2

        pipe = pltpu.emit_pipeline(
            inner,
            grid=(NB,),
            in_specs=[
                pl.BlockSpec((BB, F), lambda j: (j, 0),
                             pipeline_mode=pl.Buffered(buffer_count=3)),
                pl.BlockSpec((BB, F), lambda j: (j, 0),
                             pipeline_mode=pl.Buffered(buffer_count=2)),
                pl.BlockSpec((BB, 1), lambda j: (j, 0)),
                pl.BlockSpec((BB, 1), lambda j: (j, 0)),
            ],
            out_specs=[pl.BlockSpec((BB, 1), lambda j: (j, 0))],
        )
        pipe(white_hbm, black_hbm, turn_hbm, score_hbm, out_hbm)

    loss = pl.pallas_call(
        outer,
        in_specs=[
            pl.BlockSpec(memory_space=pl.ANY),
            pl.BlockSpec(memory_space=pl.ANY),
            pl.BlockSpec(memory_space=pl.ANY),
            pl.BlockSpec(memory_space=pl.ANY),
            pl.BlockSpec((M, F), lambda: (0, 0)),
            pl.BlockSpec(W1.shape, lambda: (0, 0)),
            pl.BlockSpec((1, N), lambda: (0, 0)),
            pl.BlockSpec((1, 2 * M), lambda: (0, 0)),
            pl.BlockSpec((1, N), lambda: (0, 0)),
            pl.BlockSpec((1, 1), lambda: (0, 0)),
        ],
        out_specs=pl.BlockSpec(memory_space=pl.ANY),
        out_shape=jax.ShapeDtypeStruct((B, 1), jnp.float32),
    )(white_features, black_features, turn, score, W0, W1, w2r, b0b0r,
      b1r_, b2r_)
    return loss


# speedup vs baseline: 1.0443x; 1.0072x over previous
import jax
import jax.numpy as jnp
from jax.experimental import pallas as pl
from jax.experimental.pallas import tpu as pltpu


def _dot_t(a, b):
    return jax.lax.dot_general(
        a, b, (((1,), (1,)), ((), ())), preferred_element_type=jnp.float32
    )


def kernel(white_features, black_features, turn, score, result, W0, b0, W1, b1, W2, b2):
    B, F = white_features.shape
    M = W0.shape[0]
    N = W1.shape[0]
    BB = 32
    NB = B // BB

    w2r = W2.reshape(1, N)
    b0b0r = jnp.concatenate([b0, b0]).reshape(1, 2 * M)
    b1r_ = b1.reshape(1, N)
    b2r_ = b2.reshape(1, 1)

    def body(white_ref, black_ref, w0_ref, w1_ref, w2_ref, b0_ref, b1_ref, b2_ref, turn_ref, score_ref, out_ref):
        j = pl.program_id(0)
        rows = pl.ds(j * BB, BB)
        wpT = _dot_t(w0_ref[...], white_ref[...])
        bpT = _dot_t(w0_ref[...], black_ref[...])
        w1 = w1_ref[...]
        w2 = w2_ref[...]
        b0b0 = b0_ref[...]
        b1r = b1_ref[...]
        b2s = b2_ref[...]
        a = jnp.concatenate([wpT, bpT], axis=0).T + b0b0
        swapped = jnp.concatenate([a[:, M:], a[:, :M]], axis=1)
        t = turn_ref[rows, :]
        accum = t * a + (1.0 - t) * swapped
        l1 = jnp.clip(accum, 0.0, 1.0)
        l2 = jnp.clip(_dot_t(l1, w1) + b1r, 0.0, 1.0)
        model_result = jnp.sum(l2 * w2, axis=1, keepdims=True) + b2s
        wdl_model = jax.nn.sigmoid(model_result / 400.0)
        wdl_target = jax.nn.sigmoid(score_ref[rows, :] / 400.0)
        out_ref[rows, :] = (wdl_model - wdl_target) ** 2

    loss = pl.pallas_call(
        body,
        grid=(NB,),
        in_specs=[
            pl.BlockSpec((BB, F), lambda j: (j, 0)),
            pl.BlockSpec((BB, F), lambda j: (j, 0)),
            pl.BlockSpec((M, F), lambda j: (0, 0)),
            pl.BlockSpec(W1.shape, lambda j: (0, 0)),
            pl.BlockSpec((1, N), lambda j: (0, 0)),
            pl.BlockSpec((1, 2 * M), lambda j: (0, 0)),
            pl.BlockSpec((1, N), lambda j: (0, 0)),
            pl.BlockSpec((1, 1), lambda j: (0, 0)),
            pl.BlockSpec((B, 1), lambda j: (0, 0)),
            pl.BlockSpec((B, 1), lambda j: (0, 0)),
        ],
        out_specs=pl.BlockSpec((B, 1), lambda j: (0, 0)),
        out_shape=jax.ShapeDtypeStruct((B, 1), jnp.float32),
        compiler_params=pltpu.CompilerParams(
            dimension_semantics=("arbitrary",),
        ),
    )(white_features, black_features, W0, W1, w2r, b0b0r, b1r_, b2r_, turn, score)
    return loss
